# baseline (speedup 1.0000x reference)
import functools

import jax
import jax.numpy as jnp
from jax import lax
from jax.experimental import pallas as pl
from jax.experimental.pallas import tpu as pltpu
from jax.experimental.pallas import tpu_sc as plsc

_D = 64
_B = 16384
_NC = 2
_NS = 16
_NW = _NC * _NS
_BPW = _B // _NW
_CH = 128
_NCH = _BPW // _CH

_mesh = plsc.VectorSubcoreMesh(core_axis_name="c", subcore_axis_name="s")


@functools.partial(
    pl.kernel,
    mesh=_mesh,
    compiler_params=pltpu.CompilerParams(use_tc_tiling_on_sc=False),
    out_type=[jax.ShapeDtypeStruct((_B, _D), jnp.float32)] * 3,
    scratch_types=[
        pltpu.VMEM((_NCH, _CH), jnp.int32),
        pltpu.VMEM((_NCH, _CH), jnp.int32),
        pltpu.VMEM((_NCH, _CH), jnp.int32),
        pltpu.VMEM((_BPW, _D), jnp.float32),
        pltpu.VMEM((_BPW, _D), jnp.float32),
        pltpu.VMEM((_BPW, _D), jnp.float32),
        pltpu.SemaphoreType.DMA,
    ],
)
def _gather3(h_idx, r_idx, t_idx, node_embs, rel_embs,
             h_out, r_out, t_out,
             h_ix, r_ix, t_ix, h_rows, r_rows, t_rows, sem):
    wid = lax.axis_index("s") * _NC + lax.axis_index("c")
    cbase = wid * _NCH
    pltpu.sync_copy(h_idx.at[pl.ds(cbase, _NCH)], h_ix)
    pltpu.sync_copy(r_idx.at[pl.ds(cbase, _NCH)], r_ix)
    pltpu.sync_copy(t_idx.at[pl.ds(cbase, _NCH)], t_ix)
    copies = []
    for j in range(_NCH):
        dst = pl.ds(j * _CH, _CH)
        copies.append(pltpu.async_copy(node_embs.at[h_ix.at[j]], h_rows.at[dst], sem))
        copies.append(pltpu.async_copy(rel_embs.at[r_ix.at[j]], r_rows.at[dst], sem))
        copies.append(pltpu.async_copy(node_embs.at[t_ix.at[j]], t_rows.at[dst], sem))
    for c in copies:
        c.wait()
    rbase = wid * _BPW
    pltpu.sync_copy(h_rows, h_out.at[pl.ds(rbase, _BPW)])
    pltpu.sync_copy(r_rows, r_out.at[pl.ds(rbase, _BPW)])
    pltpu.sync_copy(t_rows, t_out.at[pl.ds(rbase, _BPW)])


def kernel(triplets, node_embs, rel_embs):
    tri = triplets.astype(jnp.int32)
    h_idx = tri[:, 0].reshape(_B // _CH, _CH)
    r_idx = tri[:, 1].reshape(_B // _CH, _CH)
    t_idx = tri[:, 2].reshape(_B // _CH, _CH)
    head, rel, tail = _gather3(h_idx, r_idx, t_idx, node_embs, rel_embs)
    return (head, rel, tail)
